# baseline (speedup 1.0000x reference)
import functools

import jax
import jax.numpy as jnp
from jax import lax
from jax.experimental import pallas as pl

N_EMBD = 768
NUM_EXPERTS = 64
BLOCK_T = 4096
PACK_W = 16


def _router_block(x_ref, w_ref, probs_ref, pack_ref):
    x = x_ref[...]
    w = w_ref[...]
    logits = lax.dot_general(
        x, w, (((1,), (1,)), ((), ())), preferred_element_type=jnp.float32
    )
    m = jnp.max(logits, axis=1, keepdims=True)
    e = jnp.exp(logits - m)
    s = jnp.sum(e, axis=1, keepdims=True)
    probs = e / s

    iota = lax.broadcasted_iota(jnp.int32, probs.shape, 1)
    m1 = jnp.max(probs, axis=1, keepdims=True)
    i1 = jnp.min(jnp.where(probs == m1, iota, NUM_EXPERTS), axis=1, keepdims=True)
    masked = jnp.where(iota == i1, -1.0, probs)
    m2 = jnp.max(masked, axis=1, keepdims=True)
    i2 = jnp.min(jnp.where(masked == m2, iota, NUM_EXPERTS), axis=1, keepdims=True)

    denom = m1 + m2
    probs_ref[...] = probs
    pad = jnp.zeros((x.shape[0], PACK_W - 4), jnp.float32)
    pack_ref[...] = jnp.concatenate(
        [
            m1 / denom,
            m2 / denom,
            lax.bitcast_convert_type(i1, jnp.float32),
            lax.bitcast_convert_type(i2, jnp.float32),
            pad,
        ],
        axis=1,
    )


@jax.jit
def kernel(x, W):
    n_tokens = x.shape[0]
    grid = (n_tokens // BLOCK_T,)
    probs, pack = pl.pallas_call(
        _router_block,
        grid=grid,
        in_specs=[
            pl.BlockSpec((BLOCK_T, N_EMBD), lambda i: (i, 0)),
            pl.BlockSpec((NUM_EXPERTS, N_EMBD), lambda i: (0, 0)),
        ],
        out_specs=[
            pl.BlockSpec((BLOCK_T, NUM_EXPERTS), lambda i: (i, 0)),
            pl.BlockSpec((BLOCK_T, PACK_W), lambda i: (i, 0)),
        ],
        out_shape=[
            jax.ShapeDtypeStruct((n_tokens, NUM_EXPERTS), jnp.float32),
            jax.ShapeDtypeStruct((n_tokens, PACK_W), jnp.float32),
        ],
    )(x, W)
    topp = pack[:, :2]
    topi = lax.bitcast_convert_type(pack[:, 2:4], jnp.int32)
    return (topp, topi, probs)


# speedup vs baseline: 1.4016x; 1.4016x over previous
import jax
import jax.numpy as jnp
from jax import lax
from jax.experimental import pallas as pl

N_EMBD = 768
NUM_EXPERTS = 64
BLOCK_T = 4096


def _router_block(x_ref, w_ref, probs_ref, pack_ref):
    x = x_ref[...]
    w = w_ref[...]
    logits = lax.dot_general(
        x, w, (((1,), (1,)), ((), ())), preferred_element_type=jnp.float32
    )
    m = jnp.max(logits, axis=1, keepdims=True)
    e = jnp.exp(logits - m)
    s = jnp.sum(e, axis=1, keepdims=True)
    probs = e / s

    iota = lax.broadcasted_iota(jnp.int32, probs.shape, 1)
    m1 = jnp.max(probs, axis=1, keepdims=True)
    i1 = jnp.min(jnp.where(probs == m1, iota, NUM_EXPERTS), axis=1, keepdims=True)
    masked = jnp.where(iota == i1, -1.0, probs)
    m2 = jnp.max(masked, axis=1, keepdims=True)
    i2 = jnp.min(jnp.where(masked == m2, iota, NUM_EXPERTS), axis=1, keepdims=True)

    denom = m1 + m2
    probs_ref[...] = probs
    pk = jnp.concatenate(
        [
            m1 / denom,
            m2 / denom,
            lax.bitcast_convert_type(i1, jnp.float32),
            lax.bitcast_convert_type(i2, jnp.float32),
        ],
        axis=1,
    )
    pack_ref[...] = pk.T


@jax.jit
def kernel(x, W):
    n_tokens = x.shape[0]
    grid = (n_tokens // BLOCK_T,)
    probs, pack = pl.pallas_call(
        _router_block,
        grid=grid,
        in_specs=[
            pl.BlockSpec((BLOCK_T, N_EMBD), lambda i: (i, 0)),
            pl.BlockSpec((NUM_EXPERTS, N_EMBD), lambda i: (0, 0)),
        ],
        out_specs=[
            pl.BlockSpec((BLOCK_T, NUM_EXPERTS), lambda i: (i, 0)),
            pl.BlockSpec((4, BLOCK_T), lambda i: (0, i)),
        ],
        out_shape=[
            jax.ShapeDtypeStruct((n_tokens, NUM_EXPERTS), jnp.float32),
            jax.ShapeDtypeStruct((4, n_tokens), jnp.float32),
        ],
    )(x, W)
    topp = jnp.stack([pack[0], pack[1]], axis=-1)
    topi = lax.bitcast_convert_type(
        jnp.stack([pack[2], pack[3]], axis=-1), jnp.int32
    )
    return (topp, topi, probs)


# speedup vs baseline: 1.6265x; 1.1604x over previous
import jax
import jax.numpy as jnp
from jax import lax
from jax.experimental import pallas as pl

N_EMBD = 768
NUM_EXPERTS = 64
BLOCK_T = 4096


def _router_block(x_ref, w_ref, probs_ref, pack_ref):
    x = x_ref[...]
    w = w_ref[...]

    logits = lax.dot_general(
        x, w, (((1,), (1,)), ((), ())), preferred_element_type=jnp.float32
    )
    m = jnp.max(logits, axis=1, keepdims=True)
    e = jnp.exp(logits - m)
    s = jnp.sum(e, axis=1, keepdims=True)
    probs_ref[...] = e / s

    logits_t = lax.dot_general(
        w, x, (((1,), (1,)), ((), ())), preferred_element_type=jnp.float32
    )
    m_t = jnp.max(logits_t, axis=0, keepdims=True)
    e_t = jnp.exp(logits_t - m_t)

    iota = lax.broadcasted_iota(jnp.int32, e_t.shape, 0).astype(jnp.float32)
    m1 = jnp.max(e_t, axis=0, keepdims=True)
    i1 = jnp.min(
        jnp.where(e_t == m1, iota, float(NUM_EXPERTS)), axis=0, keepdims=True
    )
    masked = jnp.where(iota == i1, -1.0, e_t)
    m2 = jnp.max(masked, axis=0, keepdims=True)
    i2 = jnp.min(
        jnp.where(masked == m2, iota, float(NUM_EXPERTS)), axis=0, keepdims=True
    )

    denom = m1 + m2
    pack_ref[...] = jnp.concatenate(
        [
            m1 / denom,
            m2 / denom,
            i1,
            i2,
            jnp.zeros((4,) + m1.shape[1:], jnp.float32),
        ],
        axis=0,
    )[None]


@jax.jit
def kernel(x, W):
    n_tokens = x.shape[0]
    grid = (n_tokens // BLOCK_T,)
    probs, pack = pl.pallas_call(
        _router_block,
        grid=grid,
        in_specs=[
            pl.BlockSpec((BLOCK_T, N_EMBD), lambda i: (i, 0)),
            pl.BlockSpec((NUM_EXPERTS, N_EMBD), lambda i: (0, 0)),
        ],
        out_specs=[
            pl.BlockSpec((BLOCK_T, NUM_EXPERTS), lambda i: (i, 0)),
            pl.BlockSpec((1, 8, BLOCK_T), lambda i: (i, 0, 0)),
        ],
        out_shape=[
            jax.ShapeDtypeStruct((n_tokens, NUM_EXPERTS), jnp.float32),
            jax.ShapeDtypeStruct((grid[0], 8, BLOCK_T), jnp.float32),
        ],
    )(x, W)
    topp = jnp.stack(
        [pack[:, 0, :].reshape(n_tokens), pack[:, 1, :].reshape(n_tokens)], axis=-1
    )
    topi = jnp.stack(
        [pack[:, 2, :].reshape(n_tokens), pack[:, 3, :].reshape(n_tokens)],
        axis=-1,
    ).astype(jnp.int32)
    return (topp, topi, probs)
